# Initial kernel scaffold; baseline (speedup 1.0000x reference)
import functools

import jax
import jax.numpy as jnp
from jax import lax
from jax.experimental import pallas as pl
from jax.experimental.pallas import tpu as pltpu
from jax.experimental.pallas import tpu_sc as plsc

_N_NODES = 100000
_N = 32
_K = 16
_H = _N * _K

_L = 16
_NW = 32
_PER_TILE = 3200
_PAD = _NW * _PER_TILE
_GROUPS = _PER_TILE // _L
_UNROLL = 4

_R = 1000


def _chain_body(packed_hbm, codes_hbm, packed_v, words_v):
    c = lax.axis_index("c")
    s = lax.axis_index("s")
    wid = s * 2 + c
    base = wid * _PER_TILE
    pltpu.sync_copy(packed_hbm, packed_v)
    lanes = lax.iota(jnp.int32, _L)

    def do_group(g):
        cur = base + g * _L + lanes
        words = [jnp.zeros((_L,), jnp.int32) for _ in range(4)]
        for j in range(_K):
            v = plsc.load_gather(packed_v, [cur])
            b = jnp.where(cur != 0, (v & (_N - 1)) + 1, 0)
            words[j // 4] = words[j // 4] | (b << (8 * (j % 4)))
            cur = v >> 5
        goff = g * (_L * 4)
        for w in range(4):
            plsc.store_scatter(words_v, [goff + lanes * 4 + w], words[w])

    def body(gi, carry):
        for u in range(_UNROLL):
            do_group(gi * _UNROLL + u)
        return carry

    lax.fori_loop(0, _GROUPS // _UNROLL, body, 0)
    pltpu.sync_copy(words_v, codes_hbm.at[pl.ds(base * 4, _PER_TILE * 4)])


_chain = functools.partial(
    pl.kernel,
    out_type=jax.ShapeDtypeStruct((_PAD * 4,), jnp.int32),
    mesh=plsc.VectorSubcoreMesh(core_axis_name="c", subcore_axis_name="s"),
    scratch_types=[
        pltpu.VMEM((_PAD,), jnp.int32),
        pltpu.VMEM((_PER_TILE * 4,), jnp.int32),
    ],
)(_chain_body)


def _expand_body(codes_ref, out_ref):
    cols = lax.broadcasted_iota(jnp.int32, (_R, _N), 1) + 1
    parts = []
    for j in range(_K):
        word = codes_ref[:, j // 4:j // 4 + 1]
        b = (word >> (8 * (j % 4))) & 0xFF
        parts.append((cols == b).astype(jnp.float32))
    out_ref[...] = jnp.concatenate(parts, axis=1)


def kernel(init_embeds, parent, child_pos):
    p32 = parent.astype(jnp.int32)
    c32 = child_pos.astype(jnp.int32)
    packed = p32 * _N + c32
    packed = jnp.concatenate(
        [packed, jnp.zeros((_PAD - _N_NODES,), jnp.int32)])
    codes_flat = _chain(packed)
    codes = codes_flat.reshape(_PAD, 4)[:_N_NODES]
    out = pl.pallas_call(
        _expand_body,
        grid=(_N_NODES // _R,),
        in_specs=[pl.BlockSpec((_R, 4), lambda i: (i, 0))],
        out_specs=pl.BlockSpec((_R, _H), lambda i: (i, 0)),
        out_shape=jax.ShapeDtypeStruct((_N_NODES, _H), jnp.float32),
    )(codes)
    return out


# speedup vs baseline: 17.3581x; 17.3581x over previous
import functools

import jax
import jax.numpy as jnp
from jax import lax
from jax.experimental import pallas as pl
from jax.experimental.pallas import tpu as pltpu
from jax.experimental.pallas import tpu_sc as plsc

_N_NODES = 100000
_N = 32
_K = 16
_H = _N * _K

_L = 16
_NW = 32
_PER_TILE = 3200
_PAD = _NW * _PER_TILE
_GROUPS = _PER_TILE // _L
_UNROLL = 4

_R = 1000


def _chain_body(packed_hbm, codes_hbm, packed_v, words_v):
    c = lax.axis_index("c")
    s = lax.axis_index("s")
    wid = s * 2 + c
    base = wid * _PER_TILE
    pltpu.sync_copy(packed_hbm, packed_v)
    lanes = lax.iota(jnp.int32, _L)

    def do_group(g):
        cur = base + g * _L + lanes
        words = [jnp.zeros((_L,), jnp.int32) for _ in range(4)]
        for j in range(_K):
            v = plsc.load_gather(packed_v, [cur])
            b = jnp.where(cur != 0, (v & (_N - 1)) + 1, 0)
            words[j // 4] = words[j // 4] | (b << (8 * (j % 4)))
            cur = v >> 5
        goff = g * (_L * 4)
        for w in range(4):
            plsc.store_scatter(words_v, [goff + lanes * 4 + w], words[w])

    def body(gi, carry):
        for u in range(_UNROLL):
            do_group(gi * _UNROLL + u)
        return carry

    lax.fori_loop(0, _GROUPS // _UNROLL, body, 0)
    pltpu.sync_copy(words_v, codes_hbm.at[pl.ds(base * 4, _PER_TILE * 4)])


@functools.cache
def _chain():
    return functools.partial(
        pl.kernel,
        out_type=jax.ShapeDtypeStruct((_PAD * 4,), jnp.int32),
        mesh=plsc.VectorSubcoreMesh(core_axis_name="c", subcore_axis_name="s"),
        compiler_params=pltpu.CompilerParams(needs_layout_passes=False),
        scratch_types=[
            pltpu.VMEM((_PAD,), jnp.int32),
            pltpu.VMEM((_PER_TILE * 4,), jnp.int32),
        ],
    )(_chain_body)


def _expand_body(codes_ref, out_ref):
    cols = lax.broadcasted_iota(jnp.int32, (_R, _N), 1) + 1
    parts = []
    for j in range(_K):
        word = codes_ref[:, j // 4:j // 4 + 1]
        b = (word >> (8 * (j % 4))) & 0xFF
        parts.append((cols == b).astype(jnp.float32))
    out_ref[...] = jnp.concatenate(parts, axis=1)


def kernel(init_embeds, parent, child_pos):
    p32 = parent.astype(jnp.int32)
    c32 = child_pos.astype(jnp.int32)
    packed = p32 * _N + c32
    packed = jnp.concatenate(
        [packed, jnp.zeros((_PAD - _N_NODES,), jnp.int32)])
    codes_flat = _chain()(packed)
    codes = codes_flat.reshape(_PAD, 4)[:_N_NODES]
    out = pl.pallas_call(
        _expand_body,
        grid=(_N_NODES // _R,),
        in_specs=[pl.BlockSpec((_R, 4), lambda i: (i, 0))],
        out_specs=pl.BlockSpec((_R, _H), lambda i: (i, 0)),
        out_shape=jax.ShapeDtypeStruct((_N_NODES, _H), jnp.float32),
    )(codes)
    return out


# speedup vs baseline: 33.2171x; 1.9136x over previous
import functools

import jax
import jax.numpy as jnp
from jax import lax
from jax.experimental import pallas as pl
from jax.experimental.pallas import tpu as pltpu
from jax.experimental.pallas import tpu_sc as plsc

_N_NODES = 100000
_N = 32
_K = 16
_H = _N * _K

_L = 16
_NW = 32
_PER_TILE = 3200
_PAD = _NW * _PER_TILE
_GROUPS = _PER_TILE // _L
_UNROLL = 4

_R = 1000


def _chain_body(packed_hbm, codes_hbm, packed_v, words_v):
    c = lax.axis_index("c")
    s = lax.axis_index("s")
    wid = s * 2 + c
    base = wid * _PER_TILE
    pltpu.sync_copy(packed_hbm, packed_v)
    lanes = lax.iota(jnp.int32, _L)

    def do_group(g):
        cur = base + g * _L + lanes
        words = [jnp.zeros((_L,), jnp.int32) for _ in range(4)]
        for j in range(_K):
            v = plsc.load_gather(packed_v, [cur])
            b = jnp.where(cur != 0, (v & (_N - 1)) + 1, 0)
            words[j // 4] = words[j // 4] | (b << (8 * (j % 4)))
            cur = v >> 5
        goff = g * (_L * 4)
        for w in range(4):
            plsc.store_scatter(words_v, [goff + lanes * 4 + w], words[w])

    def body(gi, carry):
        for u in range(_UNROLL):
            do_group(gi * _UNROLL + u)
        return carry

    lax.fori_loop(0, _GROUPS // _UNROLL, body, 0)
    pltpu.sync_copy(words_v, codes_hbm.at[pl.ds(base * 4, _PER_TILE * 4)])


@functools.cache
def _chain():
    return functools.partial(
        pl.kernel,
        out_type=jax.ShapeDtypeStruct((_PAD * 4,), jnp.int32),
        mesh=plsc.VectorSubcoreMesh(core_axis_name="c", subcore_axis_name="s"),
        compiler_params=pltpu.CompilerParams(needs_layout_passes=False),
        scratch_types=[
            pltpu.VMEM((_PAD,), jnp.int32),
            pltpu.VMEM((_PER_TILE * 4,), jnp.int32),
        ],
    )(_chain_body)


def _expand_body(codes_ref, out_ref):
    lane = lax.broadcasted_iota(jnp.int32, (1, 128), 1)
    shifts = 8 * (lane >> 5)
    target = (lane & 31) + 1
    for w in range(4):
        word = codes_ref[:, w:w + 1]
        part = ((word >> shifts) & 0xFF) == target
        out_ref[:, 128 * w:128 * (w + 1)] = part.astype(jnp.float32)


def kernel(init_embeds, parent, child_pos):
    p32 = parent.astype(jnp.int32)
    c32 = child_pos.astype(jnp.int32)
    packed = p32 * _N + c32
    packed = jnp.concatenate(
        [packed, jnp.zeros((_PAD - _N_NODES,), jnp.int32)])
    codes_flat = _chain()(packed)
    codes = codes_flat.reshape(_PAD, 4)
    out = pl.pallas_call(
        _expand_body,
        grid=(_N_NODES // _R,),
        in_specs=[pl.BlockSpec((_R, 4), lambda i: (i, 0))],
        out_specs=pl.BlockSpec((_R, _H), lambda i: (i, 0)),
        out_shape=jax.ShapeDtypeStruct((_N_NODES, _H), jnp.float32),
    )(codes)
    return out


# speedup vs baseline: 37.6407x; 1.1332x over previous
import functools

import jax
import jax.numpy as jnp
from jax import lax
from jax.experimental import pallas as pl
from jax.experimental.pallas import tpu as pltpu
from jax.experimental.pallas import tpu_sc as plsc

_N_NODES = 100000
_N = 32
_K = 16
_H = _N * _K

_L = 16
_NW = 32
_PER_TILE = 3200
_PAD = _NW * _PER_TILE
_GROUPS = _PER_TILE // _L
_UNROLL = 4

_R = 2000


def _chain_body(packed_hbm, codes_hbm, packed_v, words_v):
    c = lax.axis_index("c")
    s = lax.axis_index("s")
    wid = s * 2 + c
    base = wid * _PER_TILE
    pltpu.sync_copy(packed_hbm, packed_v)
    lanes = lax.iota(jnp.int32, _L)

    def do_group(g):
        cur = base + g * _L + lanes
        words = [jnp.zeros((_L,), jnp.int32) for _ in range(4)]
        for j in range(_K):
            v = plsc.load_gather(packed_v, [cur])
            b = jnp.where(cur != 0, (v & (_N - 1)) + 1, 0)
            words[j // 4] = words[j // 4] | (b << (8 * (j % 4)))
            cur = v >> 5
        goff = g * (_L * 4)
        for w in range(4):
            plsc.store_scatter(words_v, [goff + lanes * 4 + w], words[w])

    def body(gi, carry):
        for u in range(_UNROLL):
            do_group(gi * _UNROLL + u)
        return carry

    lax.fori_loop(0, _GROUPS // _UNROLL, body, 0)
    pltpu.sync_copy(words_v, codes_hbm.at[pl.ds(base * 4, _PER_TILE * 4)])


@functools.cache
def _chain():
    return functools.partial(
        pl.kernel,
        out_type=jax.ShapeDtypeStruct((_PAD * 4,), jnp.int32),
        mesh=plsc.VectorSubcoreMesh(core_axis_name="c", subcore_axis_name="s"),
        compiler_params=pltpu.CompilerParams(needs_layout_passes=False),
        scratch_types=[
            pltpu.VMEM((_PAD,), jnp.int32),
            pltpu.VMEM((_PER_TILE * 4,), jnp.int32),
        ],
    )(_chain_body)


def _expand_body(codes_ref, out_ref):
    lane = lax.broadcasted_iota(jnp.int32, (1, 128), 1)
    shifts = 8 * (lane >> 5)
    target = (lane & 31) + 1
    for w in range(4):
        word = codes_ref[:, w:w + 1]
        part = ((word >> shifts) & 0xFF) == target
        out_ref[:, 128 * w:128 * (w + 1)] = part.astype(jnp.float32)


def kernel(init_embeds, parent, child_pos):
    p32 = parent.astype(jnp.int32)
    c32 = child_pos.astype(jnp.int32)
    packed = p32 * _N + c32
    packed = jnp.concatenate(
        [packed, jnp.zeros((_PAD - _N_NODES,), jnp.int32)])
    codes_flat = _chain()(packed)
    codes = codes_flat.reshape(_PAD, 4)
    out = pl.pallas_call(
        _expand_body,
        grid=(_N_NODES // _R,),
        in_specs=[pl.BlockSpec((_R, 4), lambda i: (i, 0))],
        out_specs=pl.BlockSpec((_R, _H), lambda i: (i, 0)),
        out_shape=jax.ShapeDtypeStruct((_N_NODES, _H), jnp.float32),
    )(codes)
    return out


# speedup vs baseline: 40.4326x; 1.0742x over previous
import functools

import jax
import jax.numpy as jnp
from jax import lax
from jax.experimental import pallas as pl
from jax.experimental.pallas import tpu as pltpu
from jax.experimental.pallas import tpu_sc as plsc

_N_NODES = 100000
_N = 32
_K = 16
_H = _N * _K

_L = 16
_NW = 32
_PER_TILE = 3200
_PAD = _NW * _PER_TILE
_GROUPS = _PER_TILE // _L
_UNROLL = 4

_R = 5000


def _chain_body(packed_hbm, codes_hbm, packed_v, words_v):
    c = lax.axis_index("c")
    s = lax.axis_index("s")
    wid = s * 2 + c
    base = wid * _PER_TILE
    pltpu.sync_copy(packed_hbm, packed_v)
    lanes = lax.iota(jnp.int32, _L)

    def do_group(g):
        cur = base + g * _L + lanes
        words = [jnp.zeros((_L,), jnp.int32) for _ in range(4)]
        for j in range(_K):
            v = plsc.load_gather(packed_v, [cur])
            b = jnp.where(cur != 0, (v & (_N - 1)) + 1, 0)
            words[j // 4] = words[j // 4] | (b << (8 * (j % 4)))
            cur = v >> 5
        goff = g * (_L * 4)
        for w in range(4):
            plsc.store_scatter(words_v, [goff + lanes * 4 + w], words[w])

    def body(gi, carry):
        for u in range(_UNROLL):
            do_group(gi * _UNROLL + u)
        return carry

    lax.fori_loop(0, _GROUPS // _UNROLL, body, 0)
    pltpu.sync_copy(words_v, codes_hbm.at[pl.ds(base * 4, _PER_TILE * 4)])


@functools.cache
def _chain():
    return functools.partial(
        pl.kernel,
        out_type=jax.ShapeDtypeStruct((_PAD * 4,), jnp.int32),
        mesh=plsc.VectorSubcoreMesh(core_axis_name="c", subcore_axis_name="s"),
        compiler_params=pltpu.CompilerParams(needs_layout_passes=False),
        scratch_types=[
            pltpu.VMEM((_PAD,), jnp.int32),
            pltpu.VMEM((_PER_TILE * 4,), jnp.int32),
        ],
    )(_chain_body)


def _expand_body(codes_ref, out_ref):
    lane = lax.broadcasted_iota(jnp.int32, (1, 128), 1)
    shifts = 8 * (lane >> 5)
    target = (lane & 31) + 1
    for w in range(4):
        word = codes_ref[:, w:w + 1]
        part = ((word >> shifts) & 0xFF) == target
        out_ref[:, 128 * w:128 * (w + 1)] = part.astype(jnp.float32)


def kernel(init_embeds, parent, child_pos):
    p32 = parent.astype(jnp.int32)
    c32 = child_pos.astype(jnp.int32)
    packed = p32 * _N + c32
    packed = jnp.concatenate(
        [packed, jnp.zeros((_PAD - _N_NODES,), jnp.int32)])
    codes_flat = _chain()(packed)
    codes = codes_flat.reshape(_PAD, 4)
    out = pl.pallas_call(
        _expand_body,
        grid=(_N_NODES // _R,),
        in_specs=[pl.BlockSpec((_R, 4), lambda i: (i, 0))],
        out_specs=pl.BlockSpec((_R, _H), lambda i: (i, 0)),
        out_shape=jax.ShapeDtypeStruct((_N_NODES, _H), jnp.float32),
    )(codes)
    return out


# speedup vs baseline: 40.5679x; 1.0033x over previous
import functools

import jax
import jax.numpy as jnp
from jax import lax
from jax.experimental import pallas as pl
from jax.experimental.pallas import tpu as pltpu
from jax.experimental.pallas import tpu_sc as plsc

_N_NODES = 100000
_N = 32
_K = 16
_H = _N * _K

_L = 16
_NW = 32
_PER_TILE = 3200
_PAD = _NW * _PER_TILE
_GROUPS = _PER_TILE // _L
_UNROLL = 4

_R = 10000


def _chain_body(packed_hbm, codes_hbm, packed_v, words_v):
    c = lax.axis_index("c")
    s = lax.axis_index("s")
    wid = s * 2 + c
    base = wid * _PER_TILE
    pltpu.sync_copy(packed_hbm, packed_v)
    lanes = lax.iota(jnp.int32, _L)

    def do_group(g):
        cur = base + g * _L + lanes
        words = [jnp.zeros((_L,), jnp.int32) for _ in range(4)]
        for j in range(_K):
            v = plsc.load_gather(packed_v, [cur])
            b = jnp.where(cur != 0, (v & (_N - 1)) + 1, 0)
            words[j // 4] = words[j // 4] | (b << (8 * (j % 4)))
            cur = v >> 5
        goff = g * (_L * 4)
        for w in range(4):
            plsc.store_scatter(words_v, [goff + lanes * 4 + w], words[w])

    def body(gi, carry):
        for u in range(_UNROLL):
            do_group(gi * _UNROLL + u)
        return carry

    lax.fori_loop(0, _GROUPS // _UNROLL, body, 0)
    pltpu.sync_copy(words_v, codes_hbm.at[pl.ds(base * 4, _PER_TILE * 4)])


@functools.cache
def _chain():
    return functools.partial(
        pl.kernel,
        out_type=jax.ShapeDtypeStruct((_PAD * 4,), jnp.int32),
        mesh=plsc.VectorSubcoreMesh(core_axis_name="c", subcore_axis_name="s"),
        compiler_params=pltpu.CompilerParams(needs_layout_passes=False),
        scratch_types=[
            pltpu.VMEM((_PAD,), jnp.int32),
            pltpu.VMEM((_PER_TILE * 4,), jnp.int32),
        ],
    )(_chain_body)


def _expand_body(codes_ref, out_ref):
    lane = lax.broadcasted_iota(jnp.int32, (1, 128), 1)
    shifts = 8 * (lane >> 5)
    target = (lane & 31) + 1
    for w in range(4):
        word = codes_ref[:, w:w + 1]
        part = ((word >> shifts) & 0xFF) == target
        out_ref[:, 128 * w:128 * (w + 1)] = part.astype(jnp.float32)


def kernel(init_embeds, parent, child_pos):
    p32 = parent.astype(jnp.int32)
    c32 = child_pos.astype(jnp.int32)
    packed = p32 * _N + c32
    packed = jnp.concatenate(
        [packed, jnp.zeros((_PAD - _N_NODES,), jnp.int32)])
    codes_flat = _chain()(packed)
    codes = codes_flat.reshape(_PAD, 4)
    out = pl.pallas_call(
        _expand_body,
        grid=(_N_NODES // _R,),
        in_specs=[pl.BlockSpec((_R, 4), lambda i: (i, 0))],
        out_specs=pl.BlockSpec((_R, _H), lambda i: (i, 0)),
        out_shape=jax.ShapeDtypeStruct((_N_NODES, _H), jnp.float32),
    )(codes)
    return out


# speedup vs baseline: 42.4206x; 1.0457x over previous
import functools

import jax
import jax.numpy as jnp
from jax import lax
from jax.experimental import pallas as pl
from jax.experimental.pallas import tpu as pltpu
from jax.experimental.pallas import tpu_sc as plsc

_N_NODES = 100000
_N = 32
_K = 16
_H = _N * _K

_L = 16
_NW = 32
_PER_TILE = 1600
_HALF = _NW * _PER_TILE
_PAD = 2 * _HALF
_SPLIT = 50000
_UNROLL = 4

_R = 5000


def _make_chain_body(row_off, table_n):
    groups = _PER_TILE // _L

    def body(packed_hbm, codes_hbm, packed_v, words_v):
        c = lax.axis_index("c")
        s = lax.axis_index("s")
        wid = s * 2 + c
        base = row_off + wid * _PER_TILE
        pltpu.sync_copy(packed_hbm.at[pl.ds(0, table_n)], packed_v)
        lanes = lax.iota(jnp.int32, _L)

        def do_group(g):
            cur = base + g * _L + lanes
            words = [jnp.zeros((_L,), jnp.int32) for _ in range(4)]
            for j in range(_K):
                v = plsc.load_gather(packed_v, [cur])
                b = jnp.where(cur != 0, (v & (_N - 1)) + 1, 0)
                words[j // 4] = words[j // 4] | (b << (8 * (j % 4)))
                cur = v >> 5
            goff = g * (_L * 4)
            for w in range(4):
                plsc.store_scatter(words_v, [goff + lanes * 4 + w], words[w])

        def loop_body(gi, carry):
            for u in range(_UNROLL):
                do_group(gi * _UNROLL + u)
            return carry

        lax.fori_loop(0, groups // _UNROLL, loop_body, 0)
        pltpu.sync_copy(
            words_v, codes_hbm.at[pl.ds(wid * (_PER_TILE * 4), _PER_TILE * 4)])

    return body


@functools.cache
def _chain(row_off, table_n):
    return functools.partial(
        pl.kernel,
        out_type=jax.ShapeDtypeStruct((_HALF * 4,), jnp.int32),
        mesh=plsc.VectorSubcoreMesh(core_axis_name="c", subcore_axis_name="s"),
        compiler_params=pltpu.CompilerParams(needs_layout_passes=False),
        scratch_types=[
            pltpu.VMEM((table_n,), jnp.int32),
            pltpu.VMEM((_PER_TILE * 4,), jnp.int32),
        ],
    )(_make_chain_body(row_off, table_n))


def _expand(codes_ref, out_ref):
    lane = lax.broadcasted_iota(jnp.int32, (1, 128), 1)
    shifts = 8 * (lane >> 5)
    target = (lane & 31) + 1
    for w in range(4):
        word = codes_ref[:, w:w + 1]
        part = ((word >> shifts) & 0xFF) == target
        out_ref[:, 128 * w:128 * (w + 1)] = part.astype(jnp.float32)


def _expand_hi(codes_ref, alias_ref, out_ref):
    del alias_ref
    _expand(codes_ref, out_ref)


def kernel(init_embeds, parent, child_pos):
    del init_embeds
    p32 = parent.astype(jnp.int32)
    c32 = child_pos.astype(jnp.int32)
    packed = p32 * _N + c32
    packed = jnp.concatenate(
        [packed, jnp.zeros((_PAD - _N_NODES,), jnp.int32)])

    codes_lo = _chain(0, _HALF)(packed).reshape(_HALF, 4)
    codes_hi = _chain(_SPLIT, _PAD)(packed).reshape(_HALF, 4)

    n_lo = _SPLIT // _R
    n_hi = (_N_NODES - _SPLIT) // _R
    out_shape = jax.ShapeDtypeStruct((_N_NODES, _H), jnp.float32)
    out_lo = pl.pallas_call(
        _expand,
        grid=(n_lo,),
        in_specs=[pl.BlockSpec((_R, 4), lambda i: (i, 0))],
        out_specs=pl.BlockSpec((_R, _H), lambda i: (i, 0)),
        out_shape=out_shape,
    )(codes_lo)
    out = pl.pallas_call(
        _expand_hi,
        grid=(n_hi,),
        in_specs=[
            pl.BlockSpec((_R, 4), lambda i: (i, 0)),
            pl.BlockSpec(memory_space=pl.ANY),
        ],
        out_specs=pl.BlockSpec((_R, _H), lambda i: (i + n_lo, 0)),
        out_shape=out_shape,
        input_output_aliases={1: 0},
    )(codes_hi, out_lo)
    return out
